# Initial kernel scaffold; baseline (speedup 1.0000x reference)
import functools

import jax
import jax.numpy as jnp
from jax import lax
from jax.experimental import pallas as pl
from jax.experimental.pallas import tpu as pltpu
from jax.experimental.pallas import tpu_sc as plsc

_N = 10000
_E = 320000
_D = 128

_NC = 2
_NS = 16
_NW = _NC * _NS
_L = 16

_C = 128
_G = _E // _C
_G_MAIN = (_G // _NW) * _NW
_RPT = _N // _NS

_mesh = plsc.VectorSubcoreMesh(
    core_axis_name="c", subcore_axis_name="s", num_cores=_NC, num_subcores=_NS
)


def _nz_partial_body(nh, eh, srcs, dsts, out, eh_v, rows_v, sidx_v, didx_v,
                     nz_s, sem):
    c = lax.axis_index("c")
    s = lax.axis_index("s")
    wid = s * _NC + c

    zero = jnp.zeros((_L,), jnp.float32)

    def zero_row(r, carry):
        for j in range(_D // _L):
            rows_v[r, pl.ds(j * _L, _L)] = zero
        return carry

    lax.fori_loop(0, _C, zero_row, 0)
    base = s * _RPT
    off = 0
    for blk in (128, 128, 128, 128, 113):
        pltpu.sync_copy(rows_v.at[pl.ds(0, blk)],
                        nz_s.at[pl.ds(base + off, blk)])
        off += blk
    plsc.subcore_barrier()

    def chunk(g):
        e0 = g * _C
        pltpu.sync_copy(srcs.at[pl.ds(e0, _C)], sidx_v)
        pltpu.sync_copy(dsts.at[pl.ds(e0, _C)], didx_v)
        pltpu.sync_copy(eh.at[pl.ds(e0, _C)], eh_v)
        pltpu.async_copy(nh.at[sidx_v], rows_v, sem).wait()

        def mul_row(r, carry):
            for j in range(_D // _L):
                sl = pl.ds(j * _L, _L)
                rows_v[r, sl] = rows_v[r, sl] * eh_v[r, sl]
            return carry

        lax.fori_loop(0, _C, mul_row, 0)
        pltpu.sync_copy(rows_v, nz_s.at[didx_v], add=True)

    def loop_body(i, carry):
        chunk(i * _NW + wid)
        return carry

    lax.fori_loop(0, _G_MAIN // _NW, loop_body, 0)
    rem = _G - _G_MAIN
    if rem:
        @pl.when(wid < rem)
        def _tail():
            chunk(_G_MAIN + wid)

    plsc.subcore_barrier()
    pltpu.sync_copy(nz_s.at[pl.ds(base, _RPT)],
                    out.at[pl.ds(c * _N + base, _RPT)])


_nz_partial = pl.kernel(
    _nz_partial_body,
    out_type=jax.ShapeDtypeStruct((2 * _N, _D), jnp.float32),
    mesh=_mesh,
    scratch_types=[
        pltpu.VMEM((_C, _D), jnp.float32),
        pltpu.VMEM((_C, _D), jnp.float32),
        pltpu.VMEM((_C,), jnp.int32),
        pltpu.VMEM((_C,), jnp.int32),
        pltpu.VMEM_SHARED((_N, _D), jnp.float32),
        pltpu.SemaphoreType.DMA,
    ],
)


def _epre_body(eh, srcs, dsts, nz, epsv, out, eh_v, srow_v, drow_v, sidx_v,
               didx_v, eps_v, sem):
    c = lax.axis_index("c")
    s = lax.axis_index("s")
    wid = s * _NC + c

    pltpu.sync_copy(epsv, eps_v)
    scale = eps_v[...] + 1.0

    def chunk(g):
        e0 = g * _C
        pltpu.sync_copy(srcs.at[pl.ds(e0, _C)], sidx_v)
        pltpu.sync_copy(dsts.at[pl.ds(e0, _C)], didx_v)
        pltpu.sync_copy(eh.at[pl.ds(e0, _C)], eh_v)
        cp1 = pltpu.async_copy(nz.at[sidx_v], srow_v, sem)
        cp2 = pltpu.async_copy(nz.at[didx_v], drow_v, sem)
        cp1.wait()
        cp2.wait()

        def row(r, carry):
            for j in range(_D // _L):
                sl = pl.ds(j * _L, _L)
                eh_v[r, sl] = (eh_v[r, sl] * scale + srow_v[r, sl]
                               - drow_v[r, sl])
            return carry

        lax.fori_loop(0, _C, row, 0)
        pltpu.sync_copy(eh_v, out.at[pl.ds(e0, _C)])

    def loop_body(i, carry):
        chunk(i * _NW + wid)
        return carry

    lax.fori_loop(0, _G_MAIN // _NW, loop_body, 0)
    rem = _G - _G_MAIN
    if rem:
        @pl.when(wid < rem)
        def _tail():
            chunk(_G_MAIN + wid)


_epre = pl.kernel(
    _epre_body,
    out_type=jax.ShapeDtypeStruct((_E, _D), jnp.float32),
    mesh=_mesh,
    scratch_types=[
        pltpu.VMEM((_C, _D), jnp.float32),
        pltpu.VMEM((_C, _D), jnp.float32),
        pltpu.VMEM((_C, _D), jnp.float32),
        pltpu.VMEM((_C,), jnp.int32),
        pltpu.VMEM((_C,), jnp.int32),
        pltpu.VMEM((_L,), jnp.float32),
        pltpu.SemaphoreType.DMA,
    ],
)


_BN = 2000


def _node_mlp_body(eps_ref, nh_ref, nz0_ref, nz1_ref, w1_ref, b1_ref, w2_ref,
                   b2_ref, nz_ref, out_ref):
    nz = nz0_ref[...] + nz1_ref[...]
    nz_ref[...] = nz
    x = (1.0 + eps_ref[0, 0]) * nh_ref[...] + nz
    h = jnp.maximum(
        jnp.dot(x, w1_ref[...], preferred_element_type=jnp.float32)
        + b1_ref[...], 0.0)
    out_ref[...] = (jnp.dot(h, w2_ref[...], preferred_element_type=jnp.float32)
                    + b2_ref[...])


def _node_mlp(eps, nh, nzp, w1, b1, w2, b2):
    nb = _N // _BN
    return pl.pallas_call(
        _node_mlp_body,
        grid=(nb,),
        in_specs=[
            pl.BlockSpec((1, 1), lambda i: (0, 0)),
            pl.BlockSpec((_BN, _D), lambda i: (i, 0)),
            pl.BlockSpec((_BN, _D), lambda i: (i, 0)),
            pl.BlockSpec((_BN, _D), lambda i: (i + nb, 0)),
            pl.BlockSpec((_D, _D), lambda i: (0, 0)),
            pl.BlockSpec((1, _D), lambda i: (0, 0)),
            pl.BlockSpec((_D, _D), lambda i: (0, 0)),
            pl.BlockSpec((1, _D), lambda i: (0, 0)),
        ],
        out_specs=[
            pl.BlockSpec((_BN, _D), lambda i: (i, 0)),
            pl.BlockSpec((_BN, _D), lambda i: (i, 0)),
        ],
        out_shape=[
            jax.ShapeDtypeStruct((_N, _D), jnp.float32),
            jax.ShapeDtypeStruct((_N, _D), jnp.float32),
        ],
    )(eps, nh, nzp, nzp, w1, b1, w2, b2)


_BE = 2500


def _edge_mlp_body(x_ref, w1_ref, b1_ref, w2_ref, b2_ref, out_ref):
    h = jnp.maximum(
        jnp.dot(x_ref[...], w1_ref[...], preferred_element_type=jnp.float32)
        + b1_ref[...], 0.0)
    out_ref[...] = (jnp.dot(h, w2_ref[...], preferred_element_type=jnp.float32)
                    + b2_ref[...])


def _edge_mlp(epre, w1, b1, w2, b2):
    return pl.pallas_call(
        _edge_mlp_body,
        grid=(_E // _BE,),
        in_specs=[
            pl.BlockSpec((_BE, _D), lambda i: (i, 0)),
            pl.BlockSpec((_D, _D), lambda i: (0, 0)),
            pl.BlockSpec((1, _D), lambda i: (0, 0)),
            pl.BlockSpec((_D, _D), lambda i: (0, 0)),
            pl.BlockSpec((1, _D), lambda i: (0, 0)),
        ],
        out_specs=pl.BlockSpec((_BE, _D), lambda i: (i, 0)),
        out_shape=jax.ShapeDtypeStruct((_E, _D), jnp.float32),
    )(epre, w1, b1, w2, b2)


def kernel(nh, eh, edge_index, nf_eps, ef_eps, nf_W1, nf_b1, nf_W2, nf_b2,
           ef_W1, ef_b1, ef_W2, ef_b2):
    src = edge_index[0].astype(jnp.int32)
    dst = edge_index[1].astype(jnp.int32)

    nzp = _nz_partial(nh, eh, src, dst)

    nz, n_h = _node_mlp(nf_eps.reshape(1, 1), nh, nzp, nf_W1,
                        nf_b1.reshape(1, _D), nf_W2, nf_b2.reshape(1, _D))

    ef_eps_vec = jnp.broadcast_to(ef_eps, (_L,))
    epre = _epre(eh, src, dst, nz, ef_eps_vec)

    e_h = _edge_mlp(epre, ef_W1, ef_b1.reshape(1, _D),
                    ef_W2, ef_b2.reshape(1, _D))
    return (n_h, e_h)


# speedup vs baseline: 2.7980x; 2.7980x over previous
import functools

import jax
import jax.numpy as jnp
from jax import lax
from jax.experimental import pallas as pl
from jax.experimental.pallas import tpu as pltpu
from jax.experimental.pallas import tpu_sc as plsc

_N = 10000
_E = 320000
_D = 128

_NC = 2
_NS = 16
_NW = _NC * _NS
_L = 16

_C = 128
_G = _E // _C
_G_MAIN = (_G // _NW) * _NW
_RPT = 624

_mesh = plsc.VectorSubcoreMesh(
    core_axis_name="c", subcore_axis_name="s", num_cores=_NC, num_subcores=_NS
)


def _nz_partial_body(nh, eh, srcs, dsts, out, eh_v, rows_v, sidx_v, didx_v,
                     nz_s, sem):
    c = lax.axis_index("c")
    s = lax.axis_index("s")
    wid = s * _NC + c

    zero = jnp.zeros((_L,), jnp.float32)

    def zero_row(r, carry):
        for j in range(_D // _L):
            rows_v[r, pl.ds(j * _L, _L)] = zero
        return carry

    lax.fori_loop(0, _C, zero_row, 0)
    base = s * _RPT
    off = 0
    for blk in (128, 128, 128, 128, 112):
        pltpu.sync_copy(rows_v.at[pl.ds(0, blk)],
                        nz_s.at[pl.ds(base + off, blk)])
        off += blk

    @pl.when(s == _NS - 1)
    def _zero_tail():
        pltpu.sync_copy(rows_v.at[pl.ds(0, _N - _NS * _RPT)],
                        nz_s.at[pl.ds(_NS * _RPT, _N - _NS * _RPT)])

    plsc.subcore_barrier()

    def chunk(g):
        e0 = g * _C
        pltpu.sync_copy(srcs.at[pl.ds(e0, _C)], sidx_v)
        pltpu.sync_copy(dsts.at[pl.ds(e0, _C)], didx_v)
        pltpu.sync_copy(eh.at[pl.ds(e0, _C)], eh_v)
        pltpu.async_copy(nh.at[sidx_v], rows_v, sem).wait()

        def mul_row(r, carry):
            for j in range(_D // _L):
                sl = pl.ds(j * _L, _L)
                rows_v[r, sl] = rows_v[r, sl] * eh_v[r, sl]
            return carry

        lax.fori_loop(0, _C, mul_row, 0)
        pltpu.sync_copy(rows_v, nz_s.at[didx_v], add=True)

    def loop_body(i, carry):
        chunk(i * _NW + wid)
        return carry

    lax.fori_loop(0, _G_MAIN // _NW, loop_body, 0)
    rem = _G - _G_MAIN
    if rem:
        @pl.when(wid < rem)
        def _tail():
            chunk(_G_MAIN + wid)

    plsc.subcore_barrier()
    pltpu.sync_copy(nz_s.at[pl.ds(base, _RPT)],
                    out.at[pl.ds(c * _N + base, _RPT)])

    @pl.when(s == _NS - 1)
    def _write_tail():
        tail = _N - _NS * _RPT
        pltpu.sync_copy(nz_s.at[pl.ds(_NS * _RPT, tail)],
                        out.at[pl.ds(c * _N + _NS * _RPT, tail)])


_nz_partial = pl.kernel(
    _nz_partial_body,
    out_type=jax.ShapeDtypeStruct((2 * _N, _D), jnp.float32),
    mesh=_mesh,
    scratch_types=[
        pltpu.VMEM((_C, _D), jnp.float32),
        pltpu.VMEM((_C, _D), jnp.float32),
        pltpu.VMEM((_C,), jnp.int32),
        pltpu.VMEM((_C,), jnp.int32),
        pltpu.VMEM_SHARED((_N, _D), jnp.float32),
        pltpu.SemaphoreType.DMA,
    ],
)


def _epre_body(eh, srcs, dsts, nz, epsv, out, eh_v, srow_v, drow_v, sidx_v,
               didx_v, eps_v, sem):
    c = lax.axis_index("c")
    s = lax.axis_index("s")
    wid = s * _NC + c

    pltpu.sync_copy(epsv, eps_v)
    scale = eps_v[...] + 1.0

    def chunk(g):
        e0 = g * _C
        pltpu.sync_copy(srcs.at[pl.ds(e0, _C)], sidx_v)
        pltpu.sync_copy(dsts.at[pl.ds(e0, _C)], didx_v)
        pltpu.sync_copy(eh.at[pl.ds(e0, _C)], eh_v)
        cp1 = pltpu.async_copy(nz.at[sidx_v], srow_v, sem)
        cp2 = pltpu.async_copy(nz.at[didx_v], drow_v, sem)
        cp1.wait()
        cp2.wait()

        def row(r, carry):
            for j in range(_D // _L):
                sl = pl.ds(j * _L, _L)
                eh_v[r, sl] = (eh_v[r, sl] * scale + srow_v[r, sl]
                               - drow_v[r, sl])
            return carry

        lax.fori_loop(0, _C, row, 0)
        pltpu.sync_copy(eh_v, out.at[pl.ds(e0, _C)])

    def loop_body(i, carry):
        chunk(i * _NW + wid)
        return carry

    lax.fori_loop(0, _G_MAIN // _NW, loop_body, 0)
    rem = _G - _G_MAIN
    if rem:
        @pl.when(wid < rem)
        def _tail():
            chunk(_G_MAIN + wid)


_epre = pl.kernel(
    _epre_body,
    out_type=jax.ShapeDtypeStruct((_E, _D), jnp.float32),
    mesh=_mesh,
    scratch_types=[
        pltpu.VMEM((_C, _D), jnp.float32),
        pltpu.VMEM((_C, _D), jnp.float32),
        pltpu.VMEM((_C, _D), jnp.float32),
        pltpu.VMEM((_C,), jnp.int32),
        pltpu.VMEM((_C,), jnp.int32),
        pltpu.VMEM((_L,), jnp.float32),
        pltpu.SemaphoreType.DMA,
    ],
)


_BN = 2000


def _node_mlp_body(eps_ref, nh_ref, nz0_ref, nz1_ref, w1_ref, b1_ref, w2_ref,
                   b2_ref, nz_ref, out_ref):
    nz = nz0_ref[...] + nz1_ref[...]
    nz_ref[...] = nz
    x = (1.0 + eps_ref[0, 0]) * nh_ref[...] + nz
    h = jnp.maximum(
        jnp.dot(x, w1_ref[...], preferred_element_type=jnp.float32)
        + b1_ref[...], 0.0)
    out_ref[...] = (jnp.dot(h, w2_ref[...], preferred_element_type=jnp.float32)
                    + b2_ref[...])


def _node_mlp(eps, nh, nzp, w1, b1, w2, b2):
    nb = _N // _BN
    return pl.pallas_call(
        _node_mlp_body,
        grid=(nb,),
        in_specs=[
            pl.BlockSpec((1, 1), lambda i: (0, 0)),
            pl.BlockSpec((_BN, _D), lambda i: (i, 0)),
            pl.BlockSpec((_BN, _D), lambda i: (i, 0)),
            pl.BlockSpec((_BN, _D), lambda i: (i + nb, 0)),
            pl.BlockSpec((_D, _D), lambda i: (0, 0)),
            pl.BlockSpec((1, _D), lambda i: (0, 0)),
            pl.BlockSpec((_D, _D), lambda i: (0, 0)),
            pl.BlockSpec((1, _D), lambda i: (0, 0)),
        ],
        out_specs=[
            pl.BlockSpec((_BN, _D), lambda i: (i, 0)),
            pl.BlockSpec((_BN, _D), lambda i: (i, 0)),
        ],
        out_shape=[
            jax.ShapeDtypeStruct((_N, _D), jnp.float32),
            jax.ShapeDtypeStruct((_N, _D), jnp.float32),
        ],
    )(eps, nh, nzp, nzp, w1, b1, w2, b2)


_BE = 2000


def _edge_mlp_body(x_ref, w1_ref, b1_ref, w2_ref, b2_ref, out_ref):
    h = jnp.maximum(
        jnp.dot(x_ref[...], w1_ref[...], preferred_element_type=jnp.float32)
        + b1_ref[...], 0.0)
    out_ref[...] = (jnp.dot(h, w2_ref[...], preferred_element_type=jnp.float32)
                    + b2_ref[...])


def _edge_mlp(epre, w1, b1, w2, b2):
    return pl.pallas_call(
        _edge_mlp_body,
        grid=(_E // _BE,),
        in_specs=[
            pl.BlockSpec((_BE, _D), lambda i: (i, 0)),
            pl.BlockSpec((_D, _D), lambda i: (0, 0)),
            pl.BlockSpec((1, _D), lambda i: (0, 0)),
            pl.BlockSpec((_D, _D), lambda i: (0, 0)),
            pl.BlockSpec((1, _D), lambda i: (0, 0)),
        ],
        out_specs=pl.BlockSpec((_BE, _D), lambda i: (i, 0)),
        out_shape=jax.ShapeDtypeStruct((_E, _D), jnp.float32),
    )(epre, w1, b1, w2, b2)


def kernel(nh, eh, edge_index, nf_eps, ef_eps, nf_W1, nf_b1, nf_W2, nf_b2,
           ef_W1, ef_b1, ef_W2, ef_b2):
    src = edge_index[0].astype(jnp.int32)
    dst = edge_index[1].astype(jnp.int32)

    nzp = _nz_partial(nh, eh, src, dst)

    nz, n_h = _node_mlp(nf_eps.reshape(1, 1), nh, nzp, nf_W1,
                        nf_b1.reshape(1, _D), nf_W2, nf_b2.reshape(1, _D))

    ef_eps_vec = jnp.broadcast_to(ef_eps, (_L,))
    epre = _epre(eh, src, dst, nz, ef_eps_vec)

    e_h = _edge_mlp(epre, ef_W1, ef_b1.reshape(1, _D),
                    ef_W2, ef_b2.reshape(1, _D))
    return (n_h, e_h)


# speedup vs baseline: 3.8676x; 1.3823x over previous
import functools

import jax
import jax.numpy as jnp
from jax import lax
from jax.experimental import pallas as pl
from jax.experimental.pallas import tpu as pltpu
from jax.experimental.pallas import tpu_sc as plsc

_N = 10000
_E = 320000
_D = 128

_NC = 2
_NS = 16
_NW = _NC * _NS
_L = 16

_C = 128
_G = _E // _C
_G_MAIN = (_G // _NW) * _NW

_CA = 64
_GA = _E // _CA
_GA_MAIN = (_GA // _NW) * _NW
_RPT = 624

_mesh = plsc.VectorSubcoreMesh(
    core_axis_name="c", subcore_axis_name="s", num_cores=_NC, num_subcores=_NS
)


def _nz_partial_body(nh, eh, srcs, dsts, out, eh_v, rows_v, sidx_v, didx_v,
                     nz_s, sem_eh, sem_g):
    c = lax.axis_index("c")
    s = lax.axis_index("s")
    wid = s * _NC + c

    zero = jnp.zeros((_L,), jnp.float32)

    def zero_row(r, carry):
        for j in range(_D // _L):
            rows_v[r, pl.ds(j * _L, _L)] = zero
        return carry

    lax.fori_loop(0, 2 * _CA, zero_row, 0)
    base = s * _RPT
    off = 0
    for blk in (128, 128, 128, 128, 112):
        pltpu.sync_copy(rows_v.at[pl.ds(0, blk)],
                        nz_s.at[pl.ds(base + off, blk)])
        off += blk

    @pl.when(s == _NS - 1)
    def _zero_tail():
        pltpu.sync_copy(rows_v.at[pl.ds(0, _N - _NS * _RPT)],
                        nz_s.at[pl.ds(_NS * _RPT, _N - _NS * _RPT)])

    plsc.subcore_barrier()

    def issue(g, slot):
        e0 = g * _CA
        pltpu.sync_copy(srcs.at[pl.ds(e0, _CA)], sidx_v.at[slot])
        pltpu.sync_copy(dsts.at[pl.ds(e0, _CA)], didx_v.at[slot])
        pltpu.async_copy(eh.at[pl.ds(e0, _CA)],
                         eh_v.at[pl.ds(slot * _CA, _CA)], sem_eh)
        pltpu.async_copy(nh.at[sidx_v.at[slot]],
                         rows_v.at[pl.ds(slot * _CA, _CA)], sem_g)

    def wait_in(slot):
        pltpu.make_async_copy(eh.at[pl.ds(0, _CA)],
                              eh_v.at[pl.ds(slot * _CA, _CA)], sem_eh).wait()
        pltpu.make_async_copy(nh.at[sidx_v.at[slot]],
                              rows_v.at[pl.ds(slot * _CA, _CA)], sem_g).wait()

    def compute_scatter(slot):
        r0 = slot * _CA

        def mul_row(r, carry):
            for j in range(_D // _L):
                sl = pl.ds(j * _L, _L)
                rows_v[r0 + r, sl] = rows_v[r0 + r, sl] * eh_v[r0 + r, sl]
            return carry

        lax.fori_loop(0, _CA, mul_row, 0)
        pltpu.sync_copy(rows_v.at[pl.ds(r0, _CA)], nz_s.at[didx_v.at[slot]],
                        add=True)

    niter = _GA_MAIN // _NW
    issue(wid, 0)

    def loop_body(k, carry):
        i0 = 2 * k
        wait_in(0)
        issue((i0 + 1) * _NW + wid, 1)
        compute_scatter(0)
        wait_in(1)

        @pl.when(i0 + 2 < niter)
        def _next():
            issue((i0 + 2) * _NW + wid, 0)

        compute_scatter(1)
        return carry

    lax.fori_loop(0, niter // 2, loop_body, 0)
    rem = _GA - _GA_MAIN
    if rem:
        @pl.when(wid < rem)
        def _tail():
            issue(_GA_MAIN + wid, 0)
            wait_in(0)
            compute_scatter(0)

    plsc.subcore_barrier()
    pltpu.sync_copy(nz_s.at[pl.ds(base, _RPT)],
                    out.at[pl.ds(c * _N + base, _RPT)])

    @pl.when(s == _NS - 1)
    def _write_tail():
        tail = _N - _NS * _RPT
        pltpu.sync_copy(nz_s.at[pl.ds(_NS * _RPT, tail)],
                        out.at[pl.ds(c * _N + _NS * _RPT, tail)])


_nz_partial = pl.kernel(
    _nz_partial_body,
    out_type=jax.ShapeDtypeStruct((2 * _N, _D), jnp.float32),
    mesh=_mesh,
    scratch_types=[
        pltpu.VMEM((2 * _CA, _D), jnp.float32),
        pltpu.VMEM((2 * _CA, _D), jnp.float32),
        pltpu.VMEM((2, _CA), jnp.int32),
        pltpu.VMEM((2, _CA), jnp.int32),
        pltpu.VMEM_SHARED((_N, _D), jnp.float32),
        pltpu.SemaphoreType.DMA,
        pltpu.SemaphoreType.DMA,
    ],
)


def _diff_body(srcs, dsts, nz, out, srow_v, drow_v, sidx_v, didx_v,
               sem_s, sem_d):
    c = lax.axis_index("c")
    s = lax.axis_index("s")
    wid = s * _NC + c

    def issue(g, slot):
        e0 = g * _C
        pltpu.sync_copy(srcs.at[pl.ds(e0, _C)], sidx_v.at[slot])
        pltpu.sync_copy(dsts.at[pl.ds(e0, _C)], didx_v.at[slot])
        pltpu.async_copy(nz.at[sidx_v.at[slot]],
                         srow_v.at[pl.ds(slot * _C, _C)], sem_s)
        pltpu.async_copy(nz.at[didx_v.at[slot]],
                         drow_v.at[pl.ds(slot * _C, _C)], sem_d)

    def wait_in(slot):
        pltpu.make_async_copy(nz.at[sidx_v.at[slot]],
                              srow_v.at[pl.ds(slot * _C, _C)], sem_s).wait()
        pltpu.make_async_copy(nz.at[didx_v.at[slot]],
                              drow_v.at[pl.ds(slot * _C, _C)], sem_d).wait()

    def compute_store(g, slot):
        r0 = slot * _C

        def row(r, carry):
            for j in range(_D // _L):
                sl = pl.ds(j * _L, _L)
                srow_v[r0 + r, sl] = srow_v[r0 + r, sl] - drow_v[r0 + r, sl]
            return carry

        lax.fori_loop(0, _C, row, 0)
        pltpu.sync_copy(srow_v.at[pl.ds(r0, _C)], out.at[pl.ds(g * _C, _C)])

    niter = _G_MAIN // _NW
    issue(wid, 0)

    def loop_body(k, carry):
        i0 = 2 * k
        wait_in(0)
        issue((i0 + 1) * _NW + wid, 1)
        compute_store(i0 * _NW + wid, 0)
        wait_in(1)

        @pl.when(i0 + 2 < niter)
        def _next():
            issue((i0 + 2) * _NW + wid, 0)

        compute_store((i0 + 1) * _NW + wid, 1)
        return carry

    lax.fori_loop(0, niter // 2, loop_body, 0)
    rem = _G - _G_MAIN
    if rem:
        @pl.when(wid < rem)
        def _tail():
            issue(_G_MAIN + wid, 0)
            wait_in(0)
            compute_store(_G_MAIN + wid, 0)


_diff = pl.kernel(
    _diff_body,
    out_type=jax.ShapeDtypeStruct((_E, _D), jnp.float32),
    mesh=_mesh,
    scratch_types=[
        pltpu.VMEM((2 * _C, _D), jnp.float32),
        pltpu.VMEM((2 * _C, _D), jnp.float32),
        pltpu.VMEM((2, _C), jnp.int32),
        pltpu.VMEM((2, _C), jnp.int32),
        pltpu.SemaphoreType.DMA,
        pltpu.SemaphoreType.DMA,
    ],
)


_BN = 2000


def _node_mlp_body(eps_ref, nh_ref, nz0_ref, nz1_ref, w1_ref, b1_ref, w2_ref,
                   b2_ref, nz_ref, out_ref):
    nz = nz0_ref[...] + nz1_ref[...]
    nz_ref[...] = nz
    x = (1.0 + eps_ref[0, 0]) * nh_ref[...] + nz
    h = jnp.maximum(
        jnp.dot(x, w1_ref[...], preferred_element_type=jnp.float32)
        + b1_ref[...], 0.0)
    out_ref[...] = (jnp.dot(h, w2_ref[...], preferred_element_type=jnp.float32)
                    + b2_ref[...])


def _node_mlp(eps, nh, nzp, w1, b1, w2, b2):
    nb = _N // _BN
    return pl.pallas_call(
        _node_mlp_body,
        grid=(nb,),
        in_specs=[
            pl.BlockSpec((1, 1), lambda i: (0, 0)),
            pl.BlockSpec((_BN, _D), lambda i: (i, 0)),
            pl.BlockSpec((_BN, _D), lambda i: (i, 0)),
            pl.BlockSpec((_BN, _D), lambda i: (i + nb, 0)),
            pl.BlockSpec((_D, _D), lambda i: (0, 0)),
            pl.BlockSpec((1, _D), lambda i: (0, 0)),
            pl.BlockSpec((_D, _D), lambda i: (0, 0)),
            pl.BlockSpec((1, _D), lambda i: (0, 0)),
        ],
        out_specs=[
            pl.BlockSpec((_BN, _D), lambda i: (i, 0)),
            pl.BlockSpec((_BN, _D), lambda i: (i, 0)),
        ],
        out_shape=[
            jax.ShapeDtypeStruct((_N, _D), jnp.float32),
            jax.ShapeDtypeStruct((_N, _D), jnp.float32),
        ],
    )(eps, nh, nzp, nzp, w1, b1, w2, b2)


_BE = 2000


def _edge_mlp_body(eps_ref, eh_ref, diff_ref, w1_ref, b1_ref, w2_ref, b2_ref,
                   out_ref):
    x = (1.0 + eps_ref[0, 0]) * eh_ref[...] + diff_ref[...]
    h = jnp.maximum(
        jnp.dot(x, w1_ref[...], preferred_element_type=jnp.float32)
        + b1_ref[...], 0.0)
    out_ref[...] = (jnp.dot(h, w2_ref[...], preferred_element_type=jnp.float32)
                    + b2_ref[...])


def _edge_mlp(eps, eh, diff, w1, b1, w2, b2):
    return pl.pallas_call(
        _edge_mlp_body,
        grid=(_E // _BE,),
        in_specs=[
            pl.BlockSpec((1, 1), lambda i: (0, 0)),
            pl.BlockSpec((_BE, _D), lambda i: (i, 0)),
            pl.BlockSpec((_BE, _D), lambda i: (i, 0)),
            pl.BlockSpec((_D, _D), lambda i: (0, 0)),
            pl.BlockSpec((1, _D), lambda i: (0, 0)),
            pl.BlockSpec((_D, _D), lambda i: (0, 0)),
            pl.BlockSpec((1, _D), lambda i: (0, 0)),
        ],
        out_specs=pl.BlockSpec((_BE, _D), lambda i: (i, 0)),
        out_shape=jax.ShapeDtypeStruct((_E, _D), jnp.float32),
    )(eps, eh, diff, w1, b1, w2, b2)


def kernel(nh, eh, edge_index, nf_eps, ef_eps, nf_W1, nf_b1, nf_W2, nf_b2,
           ef_W1, ef_b1, ef_W2, ef_b2):
    src = edge_index[0].astype(jnp.int32)
    dst = edge_index[1].astype(jnp.int32)

    nzp = _nz_partial(nh, eh, src, dst)

    nz, n_h = _node_mlp(nf_eps.reshape(1, 1), nh, nzp, nf_W1,
                        nf_b1.reshape(1, _D), nf_W2, nf_b2.reshape(1, _D))

    diff = _diff(src, dst, nz)

    e_h = _edge_mlp(ef_eps.reshape(1, 1), eh, diff, ef_W1,
                    ef_b1.reshape(1, _D), ef_W2, ef_b2.reshape(1, _D))
    return (n_h, e_h)


# speedup vs baseline: 4.0126x; 1.0375x over previous
import functools

import jax
import jax.numpy as jnp
from jax import lax
from jax.experimental import pallas as pl
from jax.experimental.pallas import tpu as pltpu
from jax.experimental.pallas import tpu_sc as plsc

_N = 10000
_E = 320000
_D = 128

_NC = 2
_NS = 16
_NW = _NC * _NS
_L = 16

_C = 128
_G = _E // _C
_G_MAIN = (_G // _NW) * _NW

_CA = 64
_GA = _E // _CA
_GA_MAIN = (_GA // _NW) * _NW
_RPT = 624

_mesh = plsc.VectorSubcoreMesh(
    core_axis_name="c", subcore_axis_name="s", num_cores=_NC, num_subcores=_NS
)


def _nz_partial_body(nh, eh, srcs, dsts, out, eh_v, rows_v, sidx_v, didx_v,
                     nz_s, sem_eh, sem_g):
    c = lax.axis_index("c")
    s = lax.axis_index("s")
    wid = s * _NC + c

    zero = jnp.zeros((_L,), jnp.float32)

    def zero_row(r, carry):
        for j in range(_D // _L):
            rows_v[r, pl.ds(j * _L, _L)] = zero
        return carry

    lax.fori_loop(0, 2 * _CA, zero_row, 0)
    base = s * _RPT
    off = 0
    for blk in (128, 128, 128, 128, 112):
        pltpu.sync_copy(rows_v.at[pl.ds(0, blk)],
                        nz_s.at[pl.ds(base + off, blk)])
        off += blk

    @pl.when(s == _NS - 1)
    def _zero_tail():
        pltpu.sync_copy(rows_v.at[pl.ds(0, _N - _NS * _RPT)],
                        nz_s.at[pl.ds(_NS * _RPT, _N - _NS * _RPT)])

    plsc.subcore_barrier()

    def issue(g, slot):
        e0 = g * _CA
        pltpu.sync_copy(srcs.at[pl.ds(e0, _CA)], sidx_v.at[slot])
        pltpu.sync_copy(dsts.at[pl.ds(e0, _CA)], didx_v.at[slot])
        pltpu.async_copy(eh.at[pl.ds(e0, _CA)],
                         eh_v.at[pl.ds(slot * _CA, _CA)], sem_eh)
        pltpu.async_copy(nh.at[sidx_v.at[slot]],
                         rows_v.at[pl.ds(slot * _CA, _CA)], sem_g)

    def wait_in(slot):
        pltpu.make_async_copy(eh.at[pl.ds(0, _CA)],
                              eh_v.at[pl.ds(slot * _CA, _CA)], sem_eh).wait()
        pltpu.make_async_copy(nh.at[sidx_v.at[slot]],
                              rows_v.at[pl.ds(slot * _CA, _CA)], sem_g).wait()

    def compute_scatter(slot):
        r0 = slot * _CA

        def mul_row(r, carry):
            for j in range(_D // _L):
                sl = pl.ds(j * _L, _L)
                rows_v[r0 + r, sl] = rows_v[r0 + r, sl] * eh_v[r0 + r, sl]
            return carry

        lax.fori_loop(0, _CA, mul_row, 0)
        pltpu.sync_copy(rows_v.at[pl.ds(r0, _CA)], nz_s.at[didx_v.at[slot]],
                        add=True)

    niter = _GA_MAIN // _NW
    issue(wid, 0)

    def loop_body(k, carry):
        i0 = 2 * k
        wait_in(0)
        issue((i0 + 1) * _NW + wid, 1)
        compute_scatter(0)
        wait_in(1)

        @pl.when(i0 + 2 < niter)
        def _next():
            issue((i0 + 2) * _NW + wid, 0)

        compute_scatter(1)
        return carry

    lax.fori_loop(0, niter // 2, loop_body, 0)
    rem = _GA - _GA_MAIN
    if rem:
        @pl.when(wid < rem)
        def _tail():
            issue(_GA_MAIN + wid, 0)
            wait_in(0)
            compute_scatter(0)

    plsc.subcore_barrier()
    pltpu.sync_copy(nz_s.at[pl.ds(base, _RPT)],
                    out.at[pl.ds(c * _N + base, _RPT)])

    @pl.when(s == _NS - 1)
    def _write_tail():
        tail = _N - _NS * _RPT
        pltpu.sync_copy(nz_s.at[pl.ds(_NS * _RPT, tail)],
                        out.at[pl.ds(c * _N + _NS * _RPT, tail)])


_nz_partial = pl.kernel(
    _nz_partial_body,
    out_type=jax.ShapeDtypeStruct((2 * _N, _D), jnp.float32),
    mesh=_mesh,
    scratch_types=[
        pltpu.VMEM((2 * _CA, _D), jnp.float32),
        pltpu.VMEM((2 * _CA, _D), jnp.float32),
        pltpu.VMEM((2, _CA), jnp.int32),
        pltpu.VMEM((2, _CA), jnp.int32),
        pltpu.VMEM_SHARED((_N, _D), jnp.float32),
        pltpu.SemaphoreType.DMA,
        pltpu.SemaphoreType.DMA,
    ],
)


def _make_diff(ep):
    gp = ep // _C
    gp_main = (gp // _NW) * _NW
    niter = gp_main // _NW
    rem = gp - gp_main

    def body(srcs, dsts, nz, out, srow_v, drow_v, obuf_v, sidx_v, didx_v,
             sem_s, sem_d, sem_o0, sem_o1):
        c = lax.axis_index("c")
        s = lax.axis_index("s")
        wid = s * _NC + c

        def issue(g, slot):
            e0 = g * _C
            pltpu.sync_copy(srcs.at[pl.ds(e0, _C)], sidx_v.at[slot])
            pltpu.sync_copy(dsts.at[pl.ds(e0, _C)], didx_v.at[slot])
            pltpu.async_copy(nz.at[sidx_v.at[slot]],
                             srow_v.at[pl.ds(slot * _C, _C)], sem_s)
            pltpu.async_copy(nz.at[didx_v.at[slot]],
                             drow_v.at[pl.ds(slot * _C, _C)], sem_d)

        def wait_in(slot):
            pltpu.make_async_copy(nz.at[sidx_v.at[slot]],
                                  srow_v.at[pl.ds(slot * _C, _C)],
                                  sem_s).wait()
            pltpu.make_async_copy(nz.at[didx_v.at[slot]],
                                  drow_v.at[pl.ds(slot * _C, _C)],
                                  sem_d).wait()

        def wait_store(slot):
            sem = sem_o0 if slot == 0 else sem_o1
            pltpu.make_async_copy(obuf_v.at[pl.ds(slot * _C, _C)],
                                  out.at[pl.ds(0, _C)], sem).wait()

        def compute_store(g, slot):
            r0 = slot * _C

            def row(r, carry):
                for j in range(_D // _L):
                    sl = pl.ds(j * _L, _L)
                    obuf_v[r0 + r, sl] = (srow_v[r0 + r, sl]
                                          - drow_v[r0 + r, sl])
                return carry

            lax.fori_loop(0, _C, row, 0)
            pltpu.async_copy(obuf_v.at[pl.ds(r0, _C)],
                             out.at[pl.ds(g * _C, _C)],
                             sem_o0 if slot == 0 else sem_o1)

        issue(wid, 0)

        def loop_body(k, carry):
            i0 = 2 * k
            wait_in(0)
            issue((i0 + 1) * _NW + wid, 1)

            @pl.when(k > 0)
            def _ws0():
                wait_store(0)

            compute_store(i0 * _NW + wid, 0)
            wait_in(1)

            @pl.when(i0 + 2 < niter)
            def _next():
                issue((i0 + 2) * _NW + wid, 0)

            @pl.when(k > 0)
            def _ws1():
                wait_store(1)

            compute_store((i0 + 1) * _NW + wid, 1)
            return carry

        lax.fori_loop(0, niter // 2, loop_body, 0)
        if niter % 2 == 1:
            wait_in(0)
            if niter > 1:
                wait_store(0)
            compute_store((niter - 1) * _NW + wid, 0)
        if rem:
            @pl.when(wid < rem)
            def _tail():
                issue(gp_main + wid, 0)
                wait_in(0)
                if niter >= 1:
                    wait_store(0)
                compute_store(gp_main + wid, 0)
        if niter >= 2:
            wait_store(1)
        if niter >= 1:
            wait_store(0)

    return pl.kernel(
        body,
        out_type=jax.ShapeDtypeStruct((ep, _D), jnp.float32),
        mesh=_mesh,
        scratch_types=[
            pltpu.VMEM((2 * _C, _D), jnp.float32),
            pltpu.VMEM((2 * _C, _D), jnp.float32),
            pltpu.VMEM((2 * _C, _D), jnp.float32),
            pltpu.VMEM((2, _C), jnp.int32),
            pltpu.VMEM((2, _C), jnp.int32),
            pltpu.SemaphoreType.DMA,
            pltpu.SemaphoreType.DMA,
            pltpu.SemaphoreType.DMA,
            pltpu.SemaphoreType.DMA,
        ],
    )


_P = 4
_EP = _E // _P
_diff_part = _make_diff(_EP)


_BN = 2000


def _node_mlp_body(eps_ref, nh_ref, nz0_ref, nz1_ref, w1_ref, b1_ref, w2_ref,
                   b2_ref, nz_ref, out_ref):
    nz = nz0_ref[...] + nz1_ref[...]
    nz_ref[...] = nz
    x = (1.0 + eps_ref[0, 0]) * nh_ref[...] + nz
    h = jnp.maximum(
        jnp.dot(x, w1_ref[...], preferred_element_type=jnp.float32)
        + b1_ref[...], 0.0)
    out_ref[...] = (jnp.dot(h, w2_ref[...], preferred_element_type=jnp.float32)
                    + b2_ref[...])


def _node_mlp(eps, nh, nzp, w1, b1, w2, b2):
    nb = _N // _BN
    return pl.pallas_call(
        _node_mlp_body,
        grid=(nb,),
        in_specs=[
            pl.BlockSpec((1, 1), lambda i: (0, 0)),
            pl.BlockSpec((_BN, _D), lambda i: (i, 0)),
            pl.BlockSpec((_BN, _D), lambda i: (i, 0)),
            pl.BlockSpec((_BN, _D), lambda i: (i + nb, 0)),
            pl.BlockSpec((_D, _D), lambda i: (0, 0)),
            pl.BlockSpec((1, _D), lambda i: (0, 0)),
            pl.BlockSpec((_D, _D), lambda i: (0, 0)),
            pl.BlockSpec((1, _D), lambda i: (0, 0)),
        ],
        out_specs=[
            pl.BlockSpec((_BN, _D), lambda i: (i, 0)),
            pl.BlockSpec((_BN, _D), lambda i: (i, 0)),
        ],
        out_shape=[
            jax.ShapeDtypeStruct((_N, _D), jnp.float32),
            jax.ShapeDtypeStruct((_N, _D), jnp.float32),
        ],
    )(eps, nh, nzp, nzp, w1, b1, w2, b2)


_BE = 2000


def _edge_mlp_part_body(acc_ref, eps_ref, eh_ref, diff_ref, w1_ref, b1_ref,
                        w2_ref, b2_ref, out_ref):
    del acc_ref
    x = (1.0 + eps_ref[0, 0]) * eh_ref[...] + diff_ref[...]
    h = jnp.maximum(
        jnp.dot(x, w1_ref[...], preferred_element_type=jnp.float32)
        + b1_ref[...], 0.0)
    out_ref[...] = (jnp.dot(h, w2_ref[...], preferred_element_type=jnp.float32)
                    + b2_ref[...])


def _edge_mlp_part(p, acc, eps, eh, diff_p, w1, b1, w2, b2):
    nbp = _EP // _BE
    return pl.pallas_call(
        _edge_mlp_part_body,
        grid=(nbp,),
        in_specs=[
            pl.BlockSpec(memory_space=pltpu.MemorySpace.HBM),
            pl.BlockSpec((1, 1), lambda i: (0, 0)),
            pl.BlockSpec((_BE, _D), lambda i, p=p: (p * nbp + i, 0)),
            pl.BlockSpec((_BE, _D), lambda i: (i, 0)),
            pl.BlockSpec((_D, _D), lambda i: (0, 0)),
            pl.BlockSpec((1, _D), lambda i: (0, 0)),
            pl.BlockSpec((_D, _D), lambda i: (0, 0)),
            pl.BlockSpec((1, _D), lambda i: (0, 0)),
        ],
        out_specs=pl.BlockSpec((_BE, _D), lambda i, p=p: (p * nbp + i, 0)),
        out_shape=jax.ShapeDtypeStruct((_E, _D), jnp.float32),
        input_output_aliases={0: 0},
    )(acc, eps, eh, diff_p, w1, b1, w2, b2)


def kernel(nh, eh, edge_index, nf_eps, ef_eps, nf_W1, nf_b1, nf_W2, nf_b2,
           ef_W1, ef_b1, ef_W2, ef_b2):
    src = edge_index[0].astype(jnp.int32)
    dst = edge_index[1].astype(jnp.int32)

    nzp = _nz_partial(nh, eh, src, dst)

    nz, n_h = _node_mlp(nf_eps.reshape(1, 1), nh, nzp, nf_W1,
                        nf_b1.reshape(1, _D), nf_W2, nf_b2.reshape(1, _D))

    ef_eps2 = ef_eps.reshape(1, 1)
    ef_b1r = ef_b1.reshape(1, _D)
    ef_b2r = ef_b2.reshape(1, _D)
    e_h = jnp.zeros((_E, _D), jnp.float32)
    for p in range(_P):
        sl = slice(p * _EP, (p + 1) * _EP)
        diff_p = _diff_part(src[sl], dst[sl], nz)
        e_h = _edge_mlp_part(p, e_h, ef_eps2, eh, diff_p, ef_W1, ef_b1r,
                             ef_W2, ef_b2r)
    return (n_h, e_h)
